# Initial kernel scaffold; baseline (speedup 1.0000x reference)
import jax
import jax.numpy as jnp
from jax.experimental import pallas as pl


def kernel(context, target, id2route, id2lr, id2prob, poi_weight, route_weight):
    raise NotImplementedError("write your pallas kernel here")


# speedup vs baseline: 1.2551x; 1.2551x over previous
import functools

import jax
import jax.numpy as jnp
from jax import lax
from jax.experimental import pallas as pl
from jax.experimental.pallas import tpu as pltpu
from jax.experimental.pallas import tpu_sc as plsc

POI = 100000
RC = 4
RD = 16
J = RC * RD
LRW = RC * (RD - 1)
D = 64
B = 4096
C = 50
NW = 32
BPW = B // NW
L = 16

_mesh = plsc.VectorSubcoreMesh(core_axis_name="c", subcore_axis_name="s")


@functools.partial(
    pl.kernel,
    out_type=jax.ShapeDtypeStruct((NW, L), jnp.float32),
    mesh=_mesh,
    compiler_params=pltpu.CompilerParams(needs_layout_passes=False,
                                         use_tc_tiling_on_sc=False),
    scratch_types=[
        pltpu.VMEM((BPW, J), jnp.int32),
        pltpu.VMEM((BPW,), jnp.int32),
        pltpu.VMEM((BPW,), jnp.int32),
        pltpu.VMEM((BPW,), jnp.int32),
        pltpu.VMEM((BPW, J), jnp.int32),
        pltpu.VMEM((BPW, 4 * LRW), jnp.int32),
        pltpu.VMEM((BPW, 4 * RC), jnp.float32),
        pltpu.VMEM((BPW, D), jnp.float32),
        pltpu.VMEM((BPW, J), jnp.float32),
        pltpu.VMEM((J, D), jnp.float32),
        pltpu.VMEM((L,), jnp.float32),
        pltpu.SemaphoreType.DMA,
    ],
)
def _poi2vec_sc(ctx_hbm, tgt_hbm, route_hbm, lr_hbm, prob_hbm, pw_hbm, rw_hbm,
                out_hbm, ctx_v, tgt_v, ptix_v, off_v, ri_v, lr_v, prob_v,
                phi_v, psi_v, rows_v, acc_v, sem):
    wid = lax.axis_index("s") * 2 + lax.axis_index("c")
    base = wid * BPW
    iota = lax.iota(jnp.int32, L)

    pltpu.sync_copy(ctx_hbm.at[pl.ds(base, BPW)], ctx_v)
    pltpu.sync_copy(tgt_hbm.at[pl.ds(base, BPW)], tgt_v)

    @pl.loop(0, BPW // L)
    def _tgt_split(kk):
        tv = tgt_v[pl.ds(kk * L, L)]
        ptix_v[pl.ds(kk * L, L)] = lax.shift_right_logical(tv, 2)
        off_v[pl.ds(kk * L, L)] = lax.bitwise_and(tv, 3) * LRW

    pltpu.async_copy(route_hbm.at[tgt_v], ri_v, sem).wait()
    pltpu.async_copy(lr_hbm.at[ptix_v], lr_v, sem).wait()
    pltpu.async_copy(prob_hbm.at[ptix_v], prob_v, sem).wait()

    @pl.loop(0, BPW)
    def _phase_a(b):
        pltpu.async_copy(pw_hbm.at[ctx_v.at[b]], rows_v, sem).wait()
        zero = jnp.zeros((L,), jnp.float32)

        @pl.loop(0, J, init_carry=(zero, zero, zero, zero), unroll=4)
        def acc(r, carry):
            a0, a1, a2, a3 = carry
            a0 = a0 + rows_v[r, pl.ds(0, L)]
            a1 = a1 + rows_v[r, pl.ds(L, L)]
            a2 = a2 + rows_v[r, pl.ds(2 * L, L)]
            a3 = a3 + rows_v[r, pl.ds(3 * L, L)]
            return a0, a1, a2, a3

        a0, a1, a2, a3 = acc
        phi_v[b, pl.ds(0, L)] = a0
        phi_v[b, pl.ds(L, L)] = a1
        phi_v[b, pl.ds(2 * L, L)] = a2
        phi_v[b, pl.ds(3 * L, L)] = a3

    @pl.loop(0, BPW)
    def _phase_b(b):
        pltpu.async_copy(rw_hbm.at[ri_v.at[b]], rows_v, sem).wait()
        bvec = jnp.full((L,), b, jnp.int32)
        offv = plsc.load_gather(off_v, [bvec])
        for t in range(RC):
            rowvec = iota + L * t
            s0 = jnp.zeros((L,), jnp.float32)

            @pl.loop(0, D // L, init_carry=s0)
            def dot(kk, s):
                pvec = phi_v[b, pl.ds(kk * L, L)]
                for j in range(L):
                    col = jnp.full((L,), kk * L + j, jnp.int32)
                    v = plsc.load_gather(rows_v, [rowvec, col])
                    s = s + v * pvec[j]
                return s

            psi = 1.0 / (1.0 + jnp.exp(-dot))
            lrcol = offv + jnp.minimum((RD - 1) * t + iota, LRW - 1)
            lrv = plsc.load_gather(lr_v, [bvec, lrcol])
            lr_eff = jnp.where(iota == L - 1, 0, lrv)
            psi_v[b, pl.ds(L * t, L)] = jnp.where(lr_eff == 1, psi, 1.0 - psi)

    bsub = lax.shift_right_logical(iota, 2)
    rc = lax.bitwise_and(iota, 3)

    def _phase_c(q, acc16):
        rows = q * 4 + bsub
        prod = jnp.ones((L,), jnp.float32)
        for d in range(RD):
            prod = prod * plsc.load_gather(psi_v, [rows, rc * L + d])
        tvec = plsc.load_gather(tgt_v, [rows])
        pv = plsc.load_gather(prob_v, [rows, lax.bitwise_and(tvec, 3) * RC + rc])
        return acc16 + prod * pv

    acc16 = lax.fori_loop(0, BPW // 4, _phase_c, jnp.zeros((L,), jnp.float32))
    acc_v[...] = acc16
    pltpu.sync_copy(acc_v, out_hbm.at[wid])


def kernel(context, target, id2route, id2lr, id2prob, poi_weight, route_weight):
    ctxp = jnp.pad(context, ((0, 0), (0, J - C)))
    route2 = id2route.reshape(POI, J)
    lr4 = id2lr.reshape(POI // 4, 4 * LRW)
    prob4 = id2prob.reshape(POI // 4, 4 * RC)
    parts = _poi2vec_sc(ctxp, target, route2, lr4, prob4,
                        poi_weight, route_weight)
    return -jnp.sum(parts) / jnp.float32(B)


# speedup vs baseline: 1.4071x; 1.1211x over previous
import functools

import jax
import jax.numpy as jnp
from jax import lax
from jax.experimental import pallas as pl
from jax.experimental.pallas import tpu as pltpu
from jax.experimental.pallas import tpu_sc as plsc

POI = 100000
RC = 4
RD = 16
J = RC * RD
LRW = RC * (RD - 1)
D = 64
B = 4096
C = 50
NW = 32
BPW = B // NW
L = 16
NCH = BPW // 2

_mesh = plsc.VectorSubcoreMesh(core_axis_name="c", subcore_axis_name="s")


@functools.partial(
    pl.kernel,
    out_type=jax.ShapeDtypeStruct((NW, L), jnp.float32),
    mesh=_mesh,
    compiler_params=pltpu.CompilerParams(needs_layout_passes=False,
                                         use_tc_tiling_on_sc=False),
    scratch_types=[
        pltpu.VMEM((NCH, 2 * J), jnp.int32),
        pltpu.VMEM((BPW,), jnp.int32),
        pltpu.VMEM((BPW,), jnp.int32),
        pltpu.VMEM((BPW,), jnp.int32),
        pltpu.VMEM((BPW, J), jnp.int32),
        pltpu.VMEM((BPW, 4 * LRW), jnp.int32),
        pltpu.VMEM((BPW, 4 * RC), jnp.float32),
        pltpu.VMEM((BPW, D), jnp.float32),
        pltpu.VMEM((BPW, J), jnp.float32),
        pltpu.VMEM((2 * J, D), jnp.float32),
        pltpu.VMEM((2 * J, D), jnp.float32),
        pltpu.VMEM((L, L), jnp.float32),
        pltpu.VMEM((L,), jnp.float32),
        pltpu.SemaphoreType.DMA,
        pltpu.SemaphoreType.DMA,
    ],
)
def _poi2vec_sc(ctx_hbm, tgt_hbm, route_hbm, lr_hbm, prob_hbm, pw_hbm, rw_hbm,
                out_hbm, ctx_v, tgt_v, ptix_v, off_v, ri_v, lr_v, prob_v,
                phi_v, psi_v, rows0, rows1, tbuf, acc_v, sem0, sem1):
    wid = lax.axis_index("s") * 2 + lax.axis_index("c")
    base = wid * BPW
    iota = lax.iota(jnp.int32, L)

    pltpu.sync_copy(ctx_hbm.at[pl.ds(wid * NCH, NCH)], ctx_v)
    pltpu.sync_copy(tgt_hbm.at[pl.ds(base, BPW)], tgt_v)

    @pl.loop(0, BPW // L)
    def _tgt_split(kk):
        tv = tgt_v[pl.ds(kk * L, L)]
        ptix_v[pl.ds(kk * L, L)] = lax.shift_right_logical(tv, 2)
        off_v[pl.ds(kk * L, L)] = lax.bitwise_and(tv, 3) * LRW

    pltpu.async_copy(route_hbm.at[tgt_v], ri_v, sem0).wait()
    pltpu.async_copy(lr_hbm.at[ptix_v], lr_v, sem0).wait()
    pltpu.async_copy(prob_hbm.at[ptix_v], prob_v, sem0).wait()

    bufs = ((rows0, sem0), (rows1, sem1))

    pltpu.async_copy(pw_hbm.at[ctx_v.at[0]], rows0, sem0)

    @pl.loop(0, NCH, step=2)
    def _phase_a(g):
        for par in range(2):
            buf, sem_cur = bufs[par]
            nbuf, sem_nxt = bufs[1 - par]
            gg = g + par

            @pl.when(gg + 1 < NCH)
            def _start_next():
                pltpu.async_copy(
                    pw_hbm.at[ctx_v.at[gg + 1]], nbuf, sem_nxt)

            pltpu.make_async_copy(
                pw_hbm.at[ctx_v.at[gg]], buf, sem_cur).wait()

            @pl.loop(0, 2)
            def _h(h):
                zero = jnp.zeros((L,), jnp.float32)

                @pl.loop(0, J, init_carry=(zero, zero, zero, zero), unroll=8)
                def acc(r, carry):
                    a0, a1, a2, a3 = carry
                    row = h * J + r
                    a0 = a0 + buf[row, pl.ds(0, L)]
                    a1 = a1 + buf[row, pl.ds(L, L)]
                    a2 = a2 + buf[row, pl.ds(2 * L, L)]
                    a3 = a3 + buf[row, pl.ds(3 * L, L)]
                    return a0, a1, a2, a3

                a0, a1, a2, a3 = acc
                b = gg * 2 + h
                phi_v[b, pl.ds(0, L)] = a0
                phi_v[b, pl.ds(L, L)] = a1
                phi_v[b, pl.ds(2 * L, L)] = a2
                phi_v[b, pl.ds(3 * L, L)] = a3

    pltpu.async_copy(rw_hbm.at[ri_v.at[0]], rows0.at[pl.ds(0, J)],
                     sem0)

    @pl.loop(0, BPW, step=2)
    def _phase_b(g):
        for par in range(2):
            buf, sem_cur = bufs[par]
            nbuf, sem_nxt = bufs[1 - par]
            b = g + par

            @pl.when(b + 1 < BPW)
            def _start_next():
                pltpu.async_copy(
                    rw_hbm.at[ri_v.at[b + 1]],
                    nbuf.at[pl.ds(0, J)], sem_nxt)

            pltpu.make_async_copy(
                rw_hbm.at[ri_v.at[b]], buf.at[pl.ds(0, J)],
                sem_cur).wait()

            @pl.loop(0, RC)
            def _bt(t):
                p0 = phi_v[b, pl.ds(0, L)]
                p1 = phi_v[b, pl.ds(L, L)]
                p2 = phi_v[b, pl.ds(2 * L, L)]
                p3 = phi_v[b, pl.ds(3 * L, L)]
                rowbase = t * L

                @pl.loop(0, L, unroll=8)
                def _j(j16):
                    row = rowbase + j16
                    pj = (buf[row, pl.ds(0, L)] * p0
                          + buf[row, pl.ds(L, L)] * p1
                          + buf[row, pl.ds(2 * L, L)] * p2
                          + buf[row, pl.ds(3 * L, L)] * p3)
                    tbuf[j16, :] = pj

                s0 = jnp.zeros((L,), jnp.float32)

                @pl.loop(0, L, init_carry=s0, unroll=8)
                def s(c, acc):
                    col = jnp.full((L,), c, jnp.int32)
                    return acc + plsc.load_gather(tbuf, [iota, col])

                psi = 1.0 / (1.0 + jnp.exp(-s))
                bvec = jnp.full((L,), b, jnp.int32)
                offv = plsc.load_gather(off_v, [bvec])
                lrcol = offv + jnp.minimum((RD - 1) * t + iota, LRW - 1)
                lrv = plsc.load_gather(lr_v, [bvec, lrcol])
                lr_eff = jnp.where(iota == L - 1, 0, lrv)
                psi_v[b, pl.ds(t * L, L)] = jnp.where(lr_eff == 1, psi,
                                                      1.0 - psi)

    bsub = lax.shift_right_logical(iota, 2)
    rc = lax.bitwise_and(iota, 3)

    def _phase_c(q, acc16):
        rows = q * 4 + bsub
        prod = jnp.ones((L,), jnp.float32)
        for d in range(RD):
            prod = prod * plsc.load_gather(psi_v, [rows, rc * L + d])
        tvec = plsc.load_gather(tgt_v, [rows])
        pv = plsc.load_gather(prob_v, [rows, lax.bitwise_and(tvec, 3) * RC + rc])
        return acc16 + prod * pv

    acc16 = lax.fori_loop(0, BPW // 4, _phase_c, jnp.zeros((L,), jnp.float32))
    acc_v[...] = acc16
    pltpu.sync_copy(acc_v, out_hbm.at[wid])


def kernel(context, target, id2route, id2lr, id2prob, poi_weight, route_weight):
    ctxp = jnp.pad(context, ((0, 0), (0, J - C))).reshape(B // 2, 2 * J)
    route2 = id2route.reshape(POI, J)
    lr4 = id2lr.reshape(POI // 4, 4 * LRW)
    prob4 = id2prob.reshape(POI // 4, 4 * RC)
    parts = _poi2vec_sc(ctxp, target, route2, lr4, prob4,
                        poi_weight, route_weight)
    return -jnp.sum(parts) / jnp.float32(B)
